# baseline (speedup 1.0000x reference)
import jax
import jax.numpy as jnp
import numpy as np
from jax.experimental import pallas as pl
from jax.experimental.pallas import tpu as pltpu

_ALPHA = 4
_NBUF = 8


def _make_body(idx, C, n):
    offs = [int(t) - _ALPHA * j for j, t in enumerate(idx)]
    tasks = [(c, j) for c in range(C) for j in range(n)]
    nt = len(tasks)

    def _body(in_hbm, fast_hbm, slow_hbm, bufs, sem_in, sem_fast, sem_slow):
        def in_dma(t):
            c, j = tasks[t]
            return pltpu.make_async_copy(
                in_hbm.at[c, pl.ds(j * _ALPHA, _ALPHA)],
                bufs.at[t % _NBUF],
                sem_in.at[t % _NBUF],
            )

        def fast_dma(t):
            c, j = tasks[t]
            return pltpu.make_async_copy(
                bufs.at[t % _NBUF],
                fast_hbm.at[c, pl.ds(j * _ALPHA, _ALPHA)],
                sem_fast.at[t % _NBUF],
            )

        def slow_dma(t):
            c, j = tasks[t]
            return pltpu.make_async_copy(
                bufs.at[t % _NBUF, pl.ds(offs[j], 1)],
                slow_hbm.at[c, pl.ds(j, 1)],
                sem_slow.at[t % _NBUF],
            )

        for t in range(min(_NBUF, nt)):
            in_dma(t).start()
        for t in range(nt):
            if t >= 1 and t - 1 + _NBUF < nt:
                fast_dma(t - 1).wait()
                slow_dma(t - 1).wait()
                in_dma(t - 1 + _NBUF).start()
            in_dma(t).wait()
            fast_dma(t).start()
            slow_dma(t).start()
        for t in range(max(0, nt - _NBUF), nt):
            fast_dma(t).wait()
            slow_dma(t).wait()

    return _body


def kernel(frames):
    C, T, H, W = frames.shape
    n = T // _ALPHA
    idx = np.linspace(0.0, T - 1, n).astype(np.int32)
    assert all(_ALPHA * j <= int(t) < _ALPHA * (j + 1) for j, t in enumerate(idx))

    fast, slow = pl.pallas_call(
        _make_body(idx, C, n),
        in_specs=[pl.BlockSpec(memory_space=pltpu.MemorySpace.HBM)],
        out_specs=[
            pl.BlockSpec(memory_space=pltpu.MemorySpace.HBM),
            pl.BlockSpec(memory_space=pltpu.MemorySpace.HBM),
        ],
        out_shape=[
            jax.ShapeDtypeStruct((C, T, H, W), frames.dtype),
            jax.ShapeDtypeStruct((C, n, H, W), frames.dtype),
        ],
        scratch_shapes=[
            pltpu.VMEM((_NBUF, _ALPHA, H, W), frames.dtype),
            pltpu.SemaphoreType.DMA((_NBUF,)),
            pltpu.SemaphoreType.DMA((_NBUF,)),
            pltpu.SemaphoreType.DMA((_NBUF,)),
        ],
    )(frames)
    return (slow, fast)


# speedup vs baseline: 1.3919x; 1.3919x over previous
import jax
import jax.numpy as jnp
import numpy as np
from jax.experimental import pallas as pl
from jax.experimental.pallas import tpu as pltpu

_ALPHA = 4


def _make_body(idx, n):
    offs = [int(t) - _ALPHA * j for j, t in enumerate(idx)]

    def _body(in_hbm, fast_hbm, slow_hbm, bufs, sem_in, sem_fast, sem_slow):
        def in_dma(j):
            return pltpu.make_async_copy(
                in_hbm.at[:, pl.ds(j * _ALPHA, _ALPHA)],
                bufs.at[j],
                sem_in.at[j],
            )

        def fast_dma(j):
            return pltpu.make_async_copy(
                bufs.at[j],
                fast_hbm.at[:, pl.ds(j * _ALPHA, _ALPHA)],
                sem_fast.at[j],
            )

        def slow_dma(j):
            return pltpu.make_async_copy(
                bufs.at[j, :, pl.ds(offs[j], 1)],
                slow_hbm.at[:, pl.ds(j, 1)],
                sem_slow.at[j],
            )

        for j in range(n):
            in_dma(j).start()
        for j in range(n):
            in_dma(j).wait()
            fast_dma(j).start()
            slow_dma(j).start()
        for j in range(n):
            fast_dma(j).wait()
            slow_dma(j).wait()

    return _body


def kernel(frames):
    C, T, H, W = frames.shape
    n = T // _ALPHA
    idx = np.linspace(0.0, T - 1, n).astype(np.int32)
    assert all(_ALPHA * j <= int(t) < _ALPHA * (j + 1) for j, t in enumerate(idx))

    fast, slow = pl.pallas_call(
        _make_body(idx, n),
        in_specs=[pl.BlockSpec(memory_space=pltpu.MemorySpace.HBM)],
        out_specs=[
            pl.BlockSpec(memory_space=pltpu.MemorySpace.HBM),
            pl.BlockSpec(memory_space=pltpu.MemorySpace.HBM),
        ],
        out_shape=[
            jax.ShapeDtypeStruct((C, T, H, W), frames.dtype),
            jax.ShapeDtypeStruct((C, n, H, W), frames.dtype),
        ],
        scratch_shapes=[
            pltpu.VMEM((n, C, _ALPHA, H, W), frames.dtype),
            pltpu.SemaphoreType.DMA((n,)),
            pltpu.SemaphoreType.DMA((n,)),
            pltpu.SemaphoreType.DMA((n,)),
        ],
    )(frames)
    return (slow, fast)
